# baseline (speedup 1.0000x reference)
import math

import jax
import jax.numpy as jnp
from jax.experimental import pallas as pl
from jax.experimental.pallas import tpu as pltpu

PHI = (1 + math.sqrt(5)) / 2
ALPHA = 1 / PHI
BETA = 1 / PHI ** 2

CH = 1024
NBUF = 2


def _hv_kernel(
    signs_hbm,
    bp_hbm,
    bm_hbm,
    out_hbm,
    signs_v,
    bp_v,
    bm_v,
    diff_ref,
    const_ref,
    obuf,
    rsems,
    ssems,
    wsems,
):
    B = signs_v.shape[0]
    L, _ = bp_v.shape
    nch = B // CH

    cp_bp = pltpu.make_async_copy(bp_hbm, bp_v, rsems.at[0])
    cp_bm = pltpu.make_async_copy(bm_hbm, bm_v, rsems.at[1])
    cp_bp.start()
    cp_bm.start()

    def signs_copy(k):
        return pltpu.make_async_copy(
            signs_hbm.at[pl.ds(k * CH, CH), :],
            signs_v.at[pl.ds(k * CH, CH), :],
            ssems.at[k],
        )

    for k in range(nch):
        signs_copy(k).start()
    cp_bp.wait()
    cp_bm.wait()

    w = jnp.where(
        jax.lax.broadcasted_iota(jnp.int32, (L, 1), 0) == 0, ALPHA, ALPHA + BETA
    )
    diff_ref[...] = ((bp_v[...] - bm_v[...]) * w).astype(jnp.bfloat16)
    const_ref[...] = jnp.sum(bm_v[...] * w, axis=0, keepdims=True)

    for k in range(nch):
        buf = k % NBUF
        if k >= NBUF:
            pltpu.make_async_copy(
                obuf.at[buf], out_hbm.at[pl.ds((k - NBUF) * CH, CH), :], wsems.at[buf]
            ).wait()
        signs_copy(k).wait()
        m = signs_v[pl.ds(k * CH, CH), :].astype(jnp.bfloat16)
        hv = (
            jnp.dot(m, diff_ref[...], preferred_element_type=jnp.float32)
            + const_ref[...]
        )
        ssq = jnp.sum(hv * hv, axis=1, keepdims=True)
        inv = jnp.where(ssq > 0, jax.lax.rsqrt(ssq), 1.0)
        obuf[buf] = hv * inv
        pltpu.make_async_copy(
            obuf.at[buf], out_hbm.at[pl.ds(k * CH, CH), :], wsems.at[buf]
        ).start()

    for k in range(max(nch - NBUF, 0), nch):
        buf = k % NBUF
        pltpu.make_async_copy(
            obuf.at[buf], out_hbm.at[pl.ds(k * CH, CH), :], wsems.at[buf]
        ).wait()


def kernel(base_plus, base_minus, signs):
    B, L = signs.shape
    D = base_plus.shape[1]
    return pl.pallas_call(
        _hv_kernel,
        in_specs=[
            pl.BlockSpec(memory_space=pl.ANY),
            pl.BlockSpec(memory_space=pl.ANY),
            pl.BlockSpec(memory_space=pl.ANY),
        ],
        out_specs=pl.BlockSpec(memory_space=pl.ANY),
        out_shape=jax.ShapeDtypeStruct((B, D), jnp.float32),
        scratch_shapes=[
            pltpu.VMEM((B, L), jnp.int32),
            pltpu.VMEM((L, D), jnp.float32),
            pltpu.VMEM((L, D), jnp.float32),
            pltpu.VMEM((L, D), jnp.bfloat16),
            pltpu.VMEM((1, D), jnp.float32),
            pltpu.VMEM((NBUF, CH, D), jnp.float32),
            pltpu.SemaphoreType.DMA((2,)),
            pltpu.SemaphoreType.DMA((B // CH,)),
            pltpu.SemaphoreType.DMA((NBUF,)),
        ],
    )(signs, base_plus, base_minus)


# speedup vs baseline: 1.0436x; 1.0436x over previous
import math

import jax
import jax.numpy as jnp
from jax.experimental import pallas as pl
from jax.experimental.pallas import tpu as pltpu

PHI = (1 + math.sqrt(5)) / 2
ALPHA = 1 / PHI
BETA = 1 / PHI ** 2

CH = 512
NBUF = 2


def _hv_kernel(
    signs_hbm,
    bp_hbm,
    bm_hbm,
    out_hbm,
    signs_v,
    bp_v,
    bm_v,
    diff_ref,
    const_ref,
    obuf,
    rsems,
    ssems,
    wsems,
):
    B = signs_v.shape[0]
    L, _ = bp_v.shape
    nch = B // CH

    cp_bp = pltpu.make_async_copy(bp_hbm, bp_v, rsems.at[0])
    cp_bm = pltpu.make_async_copy(bm_hbm, bm_v, rsems.at[1])
    cp_bp.start()
    cp_bm.start()

    def signs_copy(k):
        return pltpu.make_async_copy(
            signs_hbm.at[pl.ds(k * CH, CH), :],
            signs_v.at[pl.ds(k * CH, CH), :],
            ssems.at[k],
        )

    signs_copy(0).start()
    cp_bp.wait()
    cp_bm.wait()

    signs_copy(1).start()
    w = jnp.where(
        jax.lax.broadcasted_iota(jnp.int32, (L, 1), 0) == 0, ALPHA, ALPHA + BETA
    )
    diff_ref[...] = ((bp_v[...] - bm_v[...]) * w).astype(jnp.bfloat16)
    const_ref[...] = jnp.sum(bm_v[...] * w, axis=0, keepdims=True)

    for k in range(nch):
        if 2 <= k + 2 < nch:
            signs_copy(k + 2).start()
        buf = k % NBUF
        if k >= NBUF:
            pltpu.make_async_copy(
                obuf.at[buf], out_hbm.at[pl.ds((k - NBUF) * CH, CH), :], wsems.at[buf]
            ).wait()
        signs_copy(k).wait()
        m = signs_v[pl.ds(k * CH, CH), :].astype(jnp.bfloat16)
        hv = (
            jnp.dot(m, diff_ref[...], preferred_element_type=jnp.float32)
            + const_ref[...]
        )
        ssq = jnp.sum(hv * hv, axis=1, keepdims=True)
        inv = jnp.where(ssq > 0, jax.lax.rsqrt(ssq), 1.0)
        obuf[buf] = hv * inv
        pltpu.make_async_copy(
            obuf.at[buf], out_hbm.at[pl.ds(k * CH, CH), :], wsems.at[buf]
        ).start()

    for k in range(max(nch - NBUF, 0), nch):
        buf = k % NBUF
        pltpu.make_async_copy(
            obuf.at[buf], out_hbm.at[pl.ds(k * CH, CH), :], wsems.at[buf]
        ).wait()


def kernel(base_plus, base_minus, signs):
    B, L = signs.shape
    D = base_plus.shape[1]
    return pl.pallas_call(
        _hv_kernel,
        in_specs=[
            pl.BlockSpec(memory_space=pl.ANY),
            pl.BlockSpec(memory_space=pl.ANY),
            pl.BlockSpec(memory_space=pl.ANY),
        ],
        out_specs=pl.BlockSpec(memory_space=pl.ANY),
        out_shape=jax.ShapeDtypeStruct((B, D), jnp.float32),
        scratch_shapes=[
            pltpu.VMEM((B, L), jnp.int32),
            pltpu.VMEM((L, D), jnp.float32),
            pltpu.VMEM((L, D), jnp.float32),
            pltpu.VMEM((L, D), jnp.bfloat16),
            pltpu.VMEM((1, D), jnp.float32),
            pltpu.VMEM((NBUF, CH, D), jnp.float32),
            pltpu.SemaphoreType.DMA((2,)),
            pltpu.SemaphoreType.DMA((B // CH,)),
            pltpu.SemaphoreType.DMA((NBUF,)),
        ],
    )(signs, base_plus, base_minus)
